# Initial kernel scaffold; baseline (speedup 1.0000x reference)
import jax
import jax.numpy as jnp
from jax.experimental import pallas as pl


def kernel(x, edge_index0, edge_index1, W0, W1, bn_gamma, bn_beta, bn_mean, bn_var):
    raise NotImplementedError("write your pallas kernel here")


# speedup vs baseline: 3.8978x; 3.8978x over previous
import functools

import jax
import jax.numpy as jnp
from jax import lax
from jax.experimental import pallas as pl
from jax.experimental.pallas import tpu as pltpu
from jax.experimental.pallas import tpu_sc as plsc

N = 10000
D = 128
EPS = 1e-5

NC = 2
NS = 16
NW = NC * NS
K = 128
ROWS_PER_TILE = 624
TAIL_ROWS = N - NS * ROWS_PER_TILE
N_ACC = N + 8


def _seg_sum_partials(h, src, dst, zrows, ch):
    mesh = plsc.VectorSubcoreMesh(core_axis_name="c", subcore_axis_name="s",
                                  num_cores=NC, num_subcores=NS)

    @functools.partial(
        pl.kernel,
        out_type=jax.ShapeDtypeStruct((NC, N, D), jnp.float32),
        mesh=mesh,
        scratch_types=[
            pltpu.VMEM((K,), jnp.int32),
            pltpu.VMEM((K,), jnp.int32),
            pltpu.VMEM((K, D), jnp.float32),
            pltpu.VMEM_SHARED((N_ACC, D), jnp.float32),
            pltpu.SemaphoreType.DMA,
        ],
    )
    def k(h_hbm, src_hbm, dst_hbm, z_hbm, out_hbm, sidx, didx, rows, acc, sem):
        cid = lax.axis_index("c")
        sid = lax.axis_index("s")
        wid = cid * NS + sid
        row0 = sid * ROWS_PER_TILE

        pltpu.sync_copy(z_hbm, acc.at[pl.ds(row0, ROWS_PER_TILE)])

        @pl.when(sid == NS - 1)
        def _():
            pltpu.sync_copy(z_hbm.at[pl.ds(0, TAIL_ROWS)],
                            acc.at[pl.ds(NS * ROWS_PER_TILE, TAIL_ROWS)])

        plsc.subcore_barrier()

        base = wid * (ch * K)

        @pl.loop(0, ch)
        def _(c):
            off = base + c * K
            pltpu.sync_copy(src_hbm.at[pl.ds(off, K)], sidx)
            pltpu.sync_copy(dst_hbm.at[pl.ds(off, K)], didx)
            pltpu.async_copy(h_hbm.at[sidx], rows, sem).wait()
            pltpu.sync_copy(rows, acc.at[didx], add=True)

        plsc.subcore_barrier()
        pltpu.sync_copy(acc.at[pl.ds(row0, ROWS_PER_TILE)],
                        out_hbm.at[cid, pl.ds(row0, ROWS_PER_TILE)])

        @pl.when(sid == NS - 1)
        def _():
            pltpu.sync_copy(acc.at[pl.ds(NS * ROWS_PER_TILE, TAIL_ROWS)],
                            out_hbm.at[cid, pl.ds(NS * ROWS_PER_TILE, TAIL_ROWS)])

    return k(h, src, dst, zrows)


def _tc_matmul(x, w):
    def body(x_ref, w_ref, o_ref):
        o_ref[...] = jnp.dot(x_ref[...], w_ref[...],
                             preferred_element_type=jnp.float32,
                             precision=lax.Precision.HIGHEST)

    return pl.pallas_call(
        body, out_shape=jax.ShapeDtypeStruct((N, D), jnp.float32))(x, w)


def _tc_bn_relu_matmul(parts, gamma, beta, mean, var, w):
    def body(p_ref, g_ref, b_ref, m_ref, v_ref, w_ref, o_ref):
        s = p_ref[0] + p_ref[1]
        scale = g_ref[...] * lax.rsqrt(v_ref[...] + EPS)
        shift = b_ref[...] - m_ref[...] * scale
        y = jnp.maximum(s * scale + shift, 0.0)
        o_ref[...] = jnp.dot(y, w_ref[...],
                             preferred_element_type=jnp.float32,
                             precision=lax.Precision.HIGHEST)

    return pl.pallas_call(
        body, out_shape=jax.ShapeDtypeStruct((N, D), jnp.float32))(
            parts, gamma, beta, mean, var, w)


def _tc_log_softmax(parts):
    def body(p_ref, o_ref):
        s = p_ref[0] + p_ref[1]
        m = jnp.max(s, axis=-1, keepdims=True)
        e = jnp.exp(s - m)
        lse = jnp.log(jnp.sum(e, axis=-1, keepdims=True)) + m
        o_ref[...] = s - lse

    return pl.pallas_call(
        body, out_shape=jax.ShapeDtypeStruct((N, D), jnp.float32))(parts)


def _pad_edges(edge_index):
    e = edge_index.shape[1]
    ch = -(-e // (NW * K))
    epad = NW * ch * K
    src = edge_index[0].astype(jnp.int32)
    dst = edge_index[1].astype(jnp.int32)
    pad = epad - e
    src = jnp.concatenate([src, jnp.zeros((pad,), jnp.int32)])
    dst = jnp.concatenate([dst, jnp.full((pad,), N, jnp.int32)])
    return src, dst, ch


def kernel(x, edge_index0, edge_index1, W0, W1, bn_gamma, bn_beta, bn_mean,
           bn_var):
    x = x.astype(jnp.float32)
    zrows = jnp.zeros((ROWS_PER_TILE, D), jnp.float32)
    g = bn_gamma.reshape(1, D)
    b = bn_beta.reshape(1, D)
    m = bn_mean.reshape(1, D)
    v = bn_var.reshape(1, D)

    src0, dst0, ch0 = _pad_edges(edge_index0)
    src1, dst1, ch1 = _pad_edges(edge_index1)

    h0 = _tc_matmul(x, W0)
    p0 = _seg_sum_partials(h0, src0, dst0, zrows, ch0)
    h1 = _tc_bn_relu_matmul(p0, g, b, m, v, W1)
    p1 = _seg_sum_partials(h1, src1, dst1, zrows, ch1)
    return _tc_log_softmax(p1)
